# Initial kernel scaffold; baseline (speedup 1.0000x reference)
import jax
import jax.numpy as jnp
from jax.experimental import pallas as pl


def kernel(z, edge_index, edge_attr, batch, emb, We, be, W1, b1, W2, b2):
    raise NotImplementedError("write your pallas kernel here")


# speedup vs baseline: 3.4708x; 3.4708x over previous
import jax
import jax.numpy as jnp
from jax import lax
from jax.experimental import pallas as pl
from jax.experimental.pallas import tpu as pltpu
from jax.experimental.pallas import tpu_sc as plsc

N = 10000
H = 128
HH = 64
NE = 4
L = 3

NLANES = 16
NC = 2
NS = 16
NW = NC * NS

NPAD = 10240
ROWS_PW = NPAD // NW
EGRP = 80
NEG = ROWS_PW // EGRP

GRP = 128
NG = 158
EPS = NG * GRP
EPAD = NS * EPS
ZROWS = NPAD // NS

_mesh = plsc.VectorSubcoreMesh(core_axis_name="c", subcore_axis_name="s")

_BCAST_DNUMS = lax.GatherDimensionNumbers(
    offset_dims=(), collapsed_slice_dims=(0,), start_index_map=(0,))


def _lane_bcast(vec, t):
    idx = jnp.full((NLANES, 1), t, jnp.int32)
    return lax.gather(vec, idx, _BCAST_DNUMS, (1,),
                      mode=lax.GatherScatterMode.PROMISE_IN_BOUNDS)


def _emb_body(emb_hbm, z_hbm, out_hbm, z_v, rows_v, sem):
    c = lax.axis_index("c")
    s = lax.axis_index("s")
    wid = s * NC + c
    base = wid * ROWS_PW
    pltpu.sync_copy(z_hbm.at[pl.ds(base, ROWS_PW)], z_v)

    def body(g, carry):
        pltpu.async_copy(emb_hbm.at[z_v.at[pl.ds(g * EGRP, EGRP)]], rows_v, sem).wait()
        pltpu.sync_copy(rows_v, out_hbm.at[pl.ds(base + g * EGRP, EGRP)])
        return carry

    lax.fori_loop(0, NEG, body, 0)


def _emb_gather(emb, z_pad):
    return pl.kernel(
        _emb_body,
        out_type=jax.ShapeDtypeStruct((NPAD, H), jnp.float32),
        mesh=_mesh,
        scratch_types=[
            pltpu.VMEM((ROWS_PW,), jnp.int32),
            pltpu.VMEM((EGRP, H), jnp.float32),
            pltpu.SemaphoreType.DMA,
        ],
    )(emb, z_pad)


def _edge_body(h0_hbm, h1_hbm, src_hbm, dst_hbm, ea_hbm, we_hbm, be_hbm, zero_hbm,
               out_hbm, src_v, dst_v, ea_v, w_v, b_v, rows_v, aggr_s,
               sem_r, sem_e, sem_sc):
    c = lax.axis_index("c")
    s = lax.axis_index("s")
    ebase = s * EPS

    pltpu.sync_copy(src_hbm.at[pl.ds(ebase, EPS)], src_v)
    pltpu.sync_copy(dst_hbm.at[s], dst_v)
    pltpu.sync_copy(we_hbm.at[c], w_v)
    pltpu.sync_copy(be_hbm.at[c], b_v)

    def issue_rows(g, slot):
        idx = src_v.at[pl.ds(g * GRP, GRP)]

        @pl.when(c == 0)
        def _():
            pltpu.async_copy(h0_hbm.at[idx], rows_v.at[slot], sem_r.at[slot])

        @pl.when(c == 1)
        def _():
            pltpu.async_copy(h1_hbm.at[idx], rows_v.at[slot], sem_r.at[slot])

    def issue_ea(g, slot):
        pltpu.async_copy(ea_hbm.at[s, g], ea_v.at[slot], sem_e.at[slot])

    issue_rows(0, 0)
    issue_ea(0, 0)

    pltpu.sync_copy(zero_hbm, aggr_s.at[pl.ds(s * ZROWS, ZROWS)])
    plsc.subcore_barrier()

    W = [[w_v[k, pl.ds(j * NLANES, NLANES)] for j in range(HH // NLANES)]
         for k in range(NE)]
    B = [b_v[pl.ds(j * NLANES, NLANES)] for j in range(HH // NLANES)]

    def group(g, carry):
        slot = lax.rem(g, 2)
        nslot = 1 - slot
        pltpu.make_async_copy(h0_hbm.at[src_v.at[pl.ds(g * GRP, GRP)]],
                              rows_v.at[slot], sem_r.at[slot]).wait()
        pltpu.make_async_copy(ea_hbm.at[s, g], ea_v.at[slot], sem_e.at[slot]).wait()

        @pl.when(g >= 1)
        def _():
            pltpu.make_async_copy(rows_v.at[nslot], aggr_s.at[dst_v.at[g]],
                                  sem_sc.at[nslot]).wait()

        @pl.when(g + 1 < NG)
        def _():
            issue_rows(g + 1, nslot)
            issue_ea(g + 1, nslot)

        def block(b, bcarry):
            av = [ea_v[slot, k, pl.ds(b * NLANES, NLANES)] for k in range(NE)]
            for t in range(NLANES):
                a = [_lane_bcast(av[k], t) for k in range(NE)]
                i = b * NLANES + t
                for j in range(HH // NLANES):
                    r = rows_v[slot, i, pl.ds(j * NLANES, NLANES)]
                    e = a[0] * W[0][j] + a[1] * W[1][j] + a[2] * W[2][j] \
                        + a[3] * W[3][j] + B[j]
                    rows_v[slot, i, pl.ds(j * NLANES, NLANES)] = \
                        jnp.maximum(r + e, 0.0)
            return bcarry

        lax.fori_loop(0, GRP // NLANES, block, 0)

        pltpu.async_copy(rows_v.at[slot], aggr_s.at[dst_v.at[g]],
                         sem_sc.at[slot], add=True)
        return carry

    lax.fori_loop(0, NG, group, 0)

    pltpu.make_async_copy(rows_v.at[lax.rem(NG - 1, 2)],
                          aggr_s.at[dst_v.at[NG - 1]],
                          sem_sc.at[lax.rem(NG - 1, 2)]).wait()
    plsc.subcore_barrier()
    pltpu.sync_copy(aggr_s.at[pl.ds(s * ZROWS, ZROWS)],
                    out_hbm.at[c, pl.ds(s * ZROWS, ZROWS)])


def _edge_call(h0, h1, src_p, dst_p, ea_p, we_l, be_l, zeros):
    return pl.kernel(
        _edge_body,
        out_type=jax.ShapeDtypeStruct((NC, NPAD, HH), jnp.float32),
        mesh=_mesh,
        compiler_params=pltpu.CompilerParams(use_tc_tiling_on_sc=False),
        scratch_types=[
            pltpu.VMEM((EPS,), jnp.int32),
            pltpu.VMEM((NG, GRP), jnp.int32),
            pltpu.VMEM((2, NE, GRP), jnp.float32),
            pltpu.VMEM((NE, HH), jnp.float32),
            pltpu.VMEM((HH,), jnp.float32),
            pltpu.VMEM((2, GRP, HH), jnp.float32),
            pltpu.VMEM_SHARED((NPAD, HH), jnp.float32),
            pltpu.SemaphoreType.DMA((2,)),
            pltpu.SemaphoreType.DMA((2,)),
            pltpu.SemaphoreType.DMA((2,)),
        ],
    )(h0, h1, src_p, dst_p, ea_p, we_l, be_l, zeros)


def _node_body(h_ref, a_ref, w1_ref, b1_ref, w2_ref, b2_ref, out_ref):
    t = h_ref[...] + a_ref[...]
    u = jnp.dot(t, w1_ref[...], preferred_element_type=jnp.float32) + b1_ref[...]
    u = u * jax.nn.sigmoid(u)
    v = jnp.dot(u, w2_ref[...], preferred_element_type=jnp.float32) + b2_ref[...]
    out_ref[...] = v * jax.nn.sigmoid(v)


_node_call = pl.pallas_call(
    _node_body,
    out_shape=jax.ShapeDtypeStruct((N, H), jnp.float32),
)


def kernel(z, edge_index, edge_attr, batch, emb, We, be, W1, b1, W2, b2):
    z = z.astype(jnp.int32)
    src = edge_index[0].astype(jnp.int32)
    dst = edge_index[1].astype(jnp.int32)
    ea = edge_attr.astype(jnp.float32)

    z_pad = jnp.concatenate([z, jnp.zeros((NPAD - N,), jnp.int32)])
    src_p = jnp.concatenate([src, jnp.zeros((EPAD - src.shape[0],), jnp.int32)])
    dst_p = jnp.concatenate([dst, jnp.full((EPAD - dst.shape[0],), N, jnp.int32)])
    dst_p = dst_p.reshape(NS, NG, GRP)
    ea_p = jnp.concatenate([ea, jnp.zeros((EPAD - ea.shape[0], NE), jnp.float32)])
    ea_p = ea_p.reshape(NS, NG, GRP, NE).transpose(0, 1, 3, 2)
    zeros = jnp.zeros((ZROWS, HH), jnp.float32)

    x_pad = _emb_gather(emb, z_pad)
    h = x_pad[:N]
    for l in range(L):
        we_l = We[l].reshape(NE, NC, HH).transpose(1, 0, 2)
        be_l = be[l].reshape(NC, HH)
        h0 = h[:, :HH]
        h1 = h[:, HH:]
        aggr2 = _edge_call(h0, h1, src_p, dst_p, ea_p, we_l, be_l, zeros)
        a = jnp.concatenate([aggr2[0, :N], aggr2[1, :N]], axis=1)
        h = _node_call(h, a, W1[l], b1[l].reshape(1, H), W2[l], b2[l].reshape(1, H))
    return (h, batch)
